# baseline (speedup 1.0000x reference)
import functools

import jax
import jax.numpy as jnp
from jax import lax
from jax.experimental import pallas as pl
from jax.experimental.pallas import tpu as pltpu
from jax.experimental.pallas import tpu_sc as plsc

_NW = 10000
_NS = 10000
_NE = 160000
_D = 256
_HALF = 128
_DEGW = 16
_NTILES = 16
_EPT = _NE // _NTILES
_CHUNK = 128
_PCHUNK = 39
_NFULL = 2 * _PCHUNK
_TAIL = _EPT - _NFULL * _CHUNK
_SRCBUF = _PCHUNK * _CHUNK + _TAIL
_RPT = _NW // _NTILES

_f32 = jnp.float32


def _sc_aggregate(tabS, tabW, Erev, E, zeros):
    mesh = plsc.VectorSubcoreMesh(core_axis_name="c", subcore_axis_name="s")

    @functools.partial(
        pl.kernel,
        mesh=mesh,
        out_type=[jax.ShapeDtypeStruct((2, 2, _NW, _HALF), _f32),
                  jax.ShapeDtypeStruct((2, _NW, _DEGW), _f32)],
        scratch_types=[pltpu.VMEM((_SRCBUF,), jnp.int32),
                       pltpu.VMEM((2, _CHUNK), jnp.int32),
                       pltpu.VMEM((_TAIL,), jnp.int32),
                       pltpu.VMEM((2, _CHUNK, _HALF), _f32),
                       pltpu.VMEM((_CHUNK, _DEGW), _f32),
                       pltpu.VMEM_SHARED((_NW, _HALF), _f32),
                       pltpu.VMEM_SHARED((_NW, _DEGW), _f32),
                       pltpu.SemaphoreType.DMA,
                       pltpu.SemaphoreType.DMA,
                       pltpu.SemaphoreType.DMA,
                       pltpu.SemaphoreType.DMA],
        compiler_params=pltpu.CompilerParams(use_tc_tiling_on_sc=False),
    )
    def body(tabS_h, tabW_h, erev_h, e_h, zro_h, out_h, dout_h,
             srcv, dstv, tailv, rows, onesv, acc, dacc,
             semg, semd, sems, semo):
        c = lax.axis_index("c")
        s = lax.axis_index("s")
        rbase = s * _RPT
        ebase = s * _EPT

        onev = jnp.ones((16,), _f32)

        def fill(i, carry):
            onesv[i, pl.ds(0, 16)] = onev
            return carry

        lax.fori_loop(0, _CHUNK, fill, 0)

        for t in range(2):
            tab_c = tabS_h if t == 0 else tabW_h
            tab_h = tab_c
            edge_h = erev_h if t == 0 else e_h

            pltpu.sync_copy(zro_h, acc.at[pl.ds(rbase, _RPT)])
            pltpu.sync_copy(zro_h.at[:, pl.ds(0, _DEGW)],
                            dacc.at[pl.ds(rbase, _RPT)])
            plsc.subcore_barrier()

            def start_chunk(jloc, g, buf):
                pltpu.async_copy(
                    tab_c.at[srcv.at[pl.ds(jloc * _CHUNK, _CHUNK)]],
                    rows.at[buf], semg)
                pltpu.async_copy(
                    edge_h.at[1, pl.ds(ebase + g * _CHUNK, _CHUNK)],
                    dstv.at[buf], semd)

            def drain_chunk(buf):
                pltpu.make_async_copy(tab_h.at[pl.ds(0, _CHUNK)],
                                      rows.at[buf], semg).wait()
                pltpu.make_async_copy(edge_h.at[1, pl.ds(0, _CHUNK)],
                                      dstv.at[buf], semd).wait()

            def start_scatters(buf):
                pltpu.async_copy(rows.at[buf], acc.at[dstv.at[buf]],
                                 sems, add=True)
                pltpu.async_copy(onesv, dacc.at[dstv.at[buf]],
                                 semo, add=True)

            def drain_scatters():
                pltpu.make_async_copy(tab_h.at[pl.ds(0, _CHUNK)],
                                      rows.at[0], sems).wait()
                pltpu.make_async_copy(
                    zro_h.at[pl.ds(0, _CHUNK), pl.ds(0, _DEGW)],
                    onesv, semo).wait()

            for phase in range(2):
                gbase = phase * _PCHUNK
                npre = _PCHUNK * _CHUNK + (_TAIL if phase == 1 else 0)
                pltpu.sync_copy(
                    edge_h.at[0, pl.ds(ebase + gbase * _CHUNK, npre)],
                    srcv.at[pl.ds(0, npre)])

                def xform(i, carry):
                    v = srcv[pl.ds(i * 16, 16)]
                    srcv[pl.ds(i * 16, 16)] = v + v + c
                    return carry

                lax.fori_loop(0, npre // 16, xform, 0)

                start_chunk(0, gbase, 0)

                def pair(m, carry):
                    j = 2 * m
                    drain_chunk(0)

                    @pl.when(m > 0)
                    def _():
                        drain_scatters()

                    start_chunk(j + 1, gbase + j + 1, 1)
                    start_scatters(0)
                    drain_chunk(1)
                    drain_scatters()
                    start_chunk(j + 2, gbase + j + 2, 0)
                    start_scatters(1)
                    return carry

                lax.fori_loop(0, (_PCHUNK - 1) // 2, pair, 0)
                drain_chunk(0)
                drain_scatters()
                pltpu.sync_copy(rows.at[0], acc.at[dstv.at[0]], add=True)
                pltpu.sync_copy(onesv, dacc.at[dstv.at[0]], add=True)

            pltpu.sync_copy(
                edge_h.at[1, pl.ds(ebase + _NFULL * _CHUNK, _TAIL)], tailv)
            pltpu.async_copy(
                tab_c.at[srcv.at[pl.ds(_PCHUNK * _CHUNK, _TAIL)]],
                rows.at[0, pl.ds(0, _TAIL)], semg)
            pltpu.make_async_copy(tab_h.at[pl.ds(0, _TAIL)],
                                  rows.at[0, pl.ds(0, _TAIL)], semg).wait()
            pltpu.sync_copy(rows.at[0, pl.ds(0, _TAIL)], acc.at[tailv],
                            add=True)
            pltpu.sync_copy(onesv.at[pl.ds(0, _TAIL)], dacc.at[tailv],
                            add=True)

            plsc.subcore_barrier()
            pltpu.sync_copy(acc.at[pl.ds(rbase, _RPT)],
                            out_h.at[t, c, pl.ds(rbase, _RPT)])

            @pl.when(c == 0)
            def _():
                pltpu.sync_copy(dacc.at[pl.ds(rbase, _RPT)],
                                dout_h.at[t, pl.ds(rbase, _RPT)])
            plsc.subcore_barrier()

    return body(tabS, tabW, Erev, E, zeros)


_TCR = 1000


def _tc_body(xw, xs, aWA, aWB, aSA, aSB, dW, dS,
             W1, b1, W2, b2, Wg1, bg1, Wg2, bg2, W3, b3, W4, b4,
             outw, outs):
    def matT(x, w):
        return lax.dot_general(x, w[:], (((1,), (1,)), ((), ())),
                               preferred_element_type=_f32)

    hw = matT(xw[:], W1) + b1[:][None, :]
    hs = matT(xs[:], W2) + b2[:][None, :]

    sumS = jnp.concatenate([aWA[0, 0], aWB[0, 0]], axis=1)
    degw = dW[0, :, 0:1]
    tw = matT(matT(sumS, W2), Wg1)
    bias_w = matT(b2[:][None, :], Wg1) + bg1[:][None, :]
    nhw = (tw + degw * bias_w) / jnp.maximum(degw, 1.0)
    uw = matT(nhw + hw, W4) + b4[:][None, :]
    mw = jnp.max(uw, axis=1, keepdims=True)
    outw[:] = uw - (mw + jnp.log(jnp.sum(jnp.exp(uw - mw), axis=1,
                                         keepdims=True)))

    sumW = jnp.concatenate([aSA[0, 0], aSB[0, 0]], axis=1)
    degs = dS[0, :, 0:1]
    ts = matT(matT(sumW, W1), Wg2)
    bias_s = matT(b1[:][None, :], Wg2) + bg2[:][None, :]
    nhs = (ts + degs * bias_s) / jnp.maximum(degs, 1.0)
    us = matT(nhs + hs, W3) + b3[:][None, :]
    ms = jnp.max(us, axis=1, keepdims=True)
    outs[:] = us - (ms + jnp.log(jnp.sum(jnp.exp(us - ms), axis=1,
                                         keepdims=True)))


def _tc_dense(Xw, Xs, Agg, Deg,
              W1, b1, W2, b2, Wg1, bg1, Wg2, bg2, W3, b3, W4, b4):
    grid = (_NW // _TCR,)
    row_spec = pl.BlockSpec((_TCR, _D), lambda i: (i, 0))
    w_spec = pl.BlockSpec((_D, _D), lambda i: (0, 0))
    b_spec = pl.BlockSpec((_D,), lambda i: (0,))

    def agg_spec(t, k):
        return pl.BlockSpec((1, 1, _TCR, _HALF),
                            lambda i, t=t, k=k: (t, k, i, 0))

    def deg_spec(t):
        return pl.BlockSpec((1, _TCR, _DEGW), lambda i, t=t: (t, i, 0))

    return pl.pallas_call(
        _tc_body,
        grid=grid,
        in_specs=[row_spec, row_spec,
                  agg_spec(0, 0), agg_spec(0, 1), agg_spec(1, 0),
                  agg_spec(1, 1), deg_spec(0), deg_spec(1),
                  w_spec, b_spec, w_spec, b_spec, w_spec, b_spec,
                  w_spec, b_spec, w_spec, b_spec, w_spec, b_spec],
        out_specs=[row_spec, row_spec],
        out_shape=[jax.ShapeDtypeStruct((_NW, _D), _f32),
                   jax.ShapeDtypeStruct((_NS, _D), _f32)],
    )(Xw, Xs, Agg, Agg, Agg, Agg, Deg, Deg,
      W1, b1, W2, b2, Wg1, bg1, Wg2, bg2, W3, b3, W4, b4)


def kernel(Xw, Xs, E, Erev, W1, b1, W2, b2, Wg1, bg1, Wg2, bg2, W3, b3, W4, b4):
    tabS = Xs.reshape(2 * _NS, _HALF)
    tabW = Xw.reshape(2 * _NW, _HALF)
    zeros = jnp.zeros((_RPT, _HALF), _f32)

    Agg, Deg = _sc_aggregate(tabS, tabW, Erev.astype(jnp.int32),
                             E.astype(jnp.int32), zeros)

    ow, os_ = _tc_dense(Xw, Xs, Agg, Deg,
                        W1, b1, W2, b2, Wg1, bg1, Wg2, bg2, W3, b3, W4, b4)
    return (ow, os_)


# speedup vs baseline: 1.0120x; 1.0120x over previous
import functools

import jax
import jax.numpy as jnp
from jax import lax
from jax.experimental import pallas as pl
from jax.experimental.pallas import tpu as pltpu
from jax.experimental.pallas import tpu_sc as plsc

_NW = 10000
_NS = 10000
_NE = 160000
_D = 256
_HALF = 128
_DEGW = 16
_NTILES = 16
_EPT = _NE // _NTILES
_CHUNK = 128
_PCHUNK = 39
_NFULL = 2 * _PCHUNK
_TAIL = _EPT - _NFULL * _CHUNK
_SRCBUF = _PCHUNK * _CHUNK + _TAIL
_RPT = _NW // _NTILES

_f32 = jnp.float32


def _sc_aggregate(tabS, tabW, Erev, E, zeros):
    mesh = plsc.VectorSubcoreMesh(core_axis_name="c", subcore_axis_name="s")

    @functools.partial(
        pl.kernel,
        mesh=mesh,
        out_type=[jax.ShapeDtypeStruct((2, 2, _NW, _HALF), _f32),
                  jax.ShapeDtypeStruct((2, _NW, _DEGW), _f32)],
        scratch_types=[pltpu.VMEM((_SRCBUF,), jnp.int32),
                       pltpu.VMEM((2, _CHUNK), jnp.int32),
                       pltpu.VMEM((_TAIL,), jnp.int32),
                       pltpu.VMEM((2, _CHUNK, _HALF), _f32),
                       pltpu.VMEM((_CHUNK, _DEGW), _f32),
                       pltpu.VMEM_SHARED((_NW, _HALF), _f32),
                       pltpu.VMEM_SHARED((_NW, _DEGW), _f32),
                       pltpu.SemaphoreType.DMA,
                       pltpu.SemaphoreType.DMA,
                       pltpu.SemaphoreType.DMA,
                       pltpu.SemaphoreType.DMA],
        compiler_params=pltpu.CompilerParams(use_tc_tiling_on_sc=False),
    )
    def body(tabS_h, tabW_h, erev_h, e_h, zro_h, out_h, dout_h,
             srcv, dstv, tailv, rows, onesv, acc, dacc,
             semg, semd, sems, semo):
        c = lax.axis_index("c")
        s = lax.axis_index("s")
        rbase = s * _RPT
        ebase = s * _EPT

        onev = jnp.ones((16,), _f32)

        def fill(i, carry):
            onesv[i, pl.ds(0, 16)] = onev
            return carry

        lax.fori_loop(0, _CHUNK, fill, 0)

        for t in range(2):
            tab_h = tabS_h if t == 0 else tabW_h
            edge_h = erev_h if t == 0 else e_h
            tab_c = tab_h.at[pl.ds(c * _NW, _NW)]

            pltpu.sync_copy(zro_h, acc.at[pl.ds(rbase, _RPT)])
            pltpu.sync_copy(zro_h.at[:, pl.ds(0, _DEGW)],
                            dacc.at[pl.ds(rbase, _RPT)])
            plsc.subcore_barrier()

            def start_chunk(jloc, g, buf):
                pltpu.async_copy(
                    tab_c.at[srcv.at[pl.ds(jloc * _CHUNK, _CHUNK)]],
                    rows.at[buf], semg)
                pltpu.async_copy(
                    edge_h.at[1, pl.ds(ebase + g * _CHUNK, _CHUNK)],
                    dstv.at[buf], semd)

            def drain_chunk(buf):
                pltpu.make_async_copy(tab_h.at[pl.ds(0, _CHUNK)],
                                      rows.at[buf], semg).wait()
                pltpu.make_async_copy(edge_h.at[1, pl.ds(0, _CHUNK)],
                                      dstv.at[buf], semd).wait()

            def start_scatters(buf):
                pltpu.async_copy(rows.at[buf], acc.at[dstv.at[buf]],
                                 sems, add=True)
                pltpu.async_copy(onesv, dacc.at[dstv.at[buf]],
                                 semo, add=True)

            def drain_scatters():
                pltpu.make_async_copy(tab_h.at[pl.ds(0, _CHUNK)],
                                      rows.at[0], sems).wait()
                pltpu.make_async_copy(
                    zro_h.at[pl.ds(0, _CHUNK), pl.ds(0, _DEGW)],
                    onesv, semo).wait()

            for phase in range(2):
                gbase = phase * _PCHUNK
                npre = _PCHUNK * _CHUNK + (_TAIL if phase == 1 else 0)
                pltpu.sync_copy(
                    edge_h.at[0, pl.ds(ebase + gbase * _CHUNK, npre)],
                    srcv.at[pl.ds(0, npre)])

                start_chunk(0, gbase, 0)

                def pair(m, carry):
                    j = 2 * m
                    drain_chunk(0)

                    @pl.when(m > 0)
                    def _():
                        drain_scatters()

                    start_chunk(j + 1, gbase + j + 1, 1)
                    start_scatters(0)
                    drain_chunk(1)
                    drain_scatters()
                    start_chunk(j + 2, gbase + j + 2, 0)
                    start_scatters(1)
                    return carry

                lax.fori_loop(0, (_PCHUNK - 1) // 2, pair, 0)
                drain_chunk(0)
                drain_scatters()
                pltpu.sync_copy(rows.at[0], acc.at[dstv.at[0]], add=True)
                pltpu.sync_copy(onesv, dacc.at[dstv.at[0]], add=True)

            pltpu.sync_copy(
                edge_h.at[1, pl.ds(ebase + _NFULL * _CHUNK, _TAIL)], tailv)
            pltpu.async_copy(
                tab_c.at[srcv.at[pl.ds(_PCHUNK * _CHUNK, _TAIL)]],
                rows.at[0, pl.ds(0, _TAIL)], semg)
            pltpu.make_async_copy(tab_h.at[pl.ds(0, _TAIL)],
                                  rows.at[0, pl.ds(0, _TAIL)], semg).wait()
            pltpu.sync_copy(rows.at[0, pl.ds(0, _TAIL)], acc.at[tailv],
                            add=True)
            pltpu.sync_copy(onesv.at[pl.ds(0, _TAIL)], dacc.at[tailv],
                            add=True)

            plsc.subcore_barrier()
            pltpu.sync_copy(acc.at[pl.ds(rbase, _RPT)],
                            out_h.at[t, c, pl.ds(rbase, _RPT)])

            @pl.when(c == 0)
            def _():
                pltpu.sync_copy(dacc.at[pl.ds(rbase, _RPT)],
                                dout_h.at[t, pl.ds(rbase, _RPT)])
            plsc.subcore_barrier()

    return body(tabS, tabW, Erev, E, zeros)


_TCR = 2000


def _tc_body(xw, xs, aWA, aWB, aSA, aSB, dW, dS,
             W1, b1, W2, b2, Wg1, bg1, Wg2, bg2, W3, b3, W4, b4,
             outw, outs):
    def matT(x, w):
        return lax.dot_general(x, w[:], (((1,), (1,)), ((), ())),
                               preferred_element_type=_f32)

    hw = matT(xw[:], W1) + b1[:][None, :]
    hs = matT(xs[:], W2) + b2[:][None, :]

    sumS = jnp.concatenate([aWA[0, 0], aWB[0, 0]], axis=1)
    degw = dW[0, :, 0:1]
    tw = matT(matT(sumS, W2), Wg1)
    bias_w = matT(b2[:][None, :], Wg1) + bg1[:][None, :]
    nhw = (tw + degw * bias_w) / jnp.maximum(degw, 1.0)
    uw = matT(nhw + hw, W4) + b4[:][None, :]
    mw = jnp.max(uw, axis=1, keepdims=True)
    outw[:] = uw - (mw + jnp.log(jnp.sum(jnp.exp(uw - mw), axis=1,
                                         keepdims=True)))

    sumW = jnp.concatenate([aSA[0, 0], aSB[0, 0]], axis=1)
    degs = dS[0, :, 0:1]
    ts = matT(matT(sumW, W1), Wg2)
    bias_s = matT(b1[:][None, :], Wg2) + bg2[:][None, :]
    nhs = (ts + degs * bias_s) / jnp.maximum(degs, 1.0)
    us = matT(nhs + hs, W3) + b3[:][None, :]
    ms = jnp.max(us, axis=1, keepdims=True)
    outs[:] = us - (ms + jnp.log(jnp.sum(jnp.exp(us - ms), axis=1,
                                         keepdims=True)))


def _tc_dense(Xw, Xs, Agg, Deg,
              W1, b1, W2, b2, Wg1, bg1, Wg2, bg2, W3, b3, W4, b4):
    grid = (_NW // _TCR,)
    row_spec = pl.BlockSpec((_TCR, _D), lambda i: (i, 0))
    w_spec = pl.BlockSpec((_D, _D), lambda i: (0, 0))
    b_spec = pl.BlockSpec((_D,), lambda i: (0,))

    def agg_spec(t, k):
        return pl.BlockSpec((1, 1, _TCR, _HALF),
                            lambda i, t=t, k=k: (t, k, i, 0))

    def deg_spec(t):
        return pl.BlockSpec((1, _TCR, _DEGW), lambda i, t=t: (t, i, 0))

    return pl.pallas_call(
        _tc_body,
        grid=grid,
        in_specs=[row_spec, row_spec,
                  agg_spec(0, 0), agg_spec(0, 1), agg_spec(1, 0),
                  agg_spec(1, 1), deg_spec(0), deg_spec(1),
                  w_spec, b_spec, w_spec, b_spec, w_spec, b_spec,
                  w_spec, b_spec, w_spec, b_spec, w_spec, b_spec],
        out_specs=[row_spec, row_spec],
        out_shape=[jax.ShapeDtypeStruct((_NW, _D), _f32),
                   jax.ShapeDtypeStruct((_NS, _D), _f32)],
    )(Xw, Xs, Agg, Agg, Agg, Agg, Deg, Deg,
      W1, b1, W2, b2, Wg1, bg1, Wg2, bg2, W3, b3, W4, b4)


def kernel(Xw, Xs, E, Erev, W1, b1, W2, b2, Wg1, bg1, Wg2, bg2, W3, b3, W4, b4):
    tabS = jnp.concatenate([Xs[:, :_HALF], Xs[:, _HALF:]], axis=0)
    tabW = jnp.concatenate([Xw[:, :_HALF], Xw[:, _HALF:]], axis=0)
    zeros = jnp.zeros((_RPT, _HALF), _f32)

    Agg, Deg = _sc_aggregate(tabS, tabW, Erev.astype(jnp.int32),
                             E.astype(jnp.int32), zeros)

    ow, os_ = _tc_dense(Xw, Xs, Agg, Deg,
                        W1, b1, W2, b2, Wg1, bg1, Wg2, bg2, W3, b3, W4, b4)
    return (ow, os_)


# speedup vs baseline: 1.0131x; 1.0011x over previous
import functools

import jax
import jax.numpy as jnp
from jax import lax
from jax.experimental import pallas as pl
from jax.experimental.pallas import tpu as pltpu
from jax.experimental.pallas import tpu_sc as plsc

_NW = 10000
_NS = 10000
_NE = 160000
_D = 256
_HALF = 128
_DEGW = 16
_NTILES = 16
_EPT = _NE // _NTILES
_CHUNK = 128
_PCHUNK = 39
_NFULL = 2 * _PCHUNK
_TAIL = _EPT - _NFULL * _CHUNK
_SRCBUF = _PCHUNK * _CHUNK + _TAIL
_RPT = _NW // _NTILES

_f32 = jnp.float32


def _sc_aggregate(tabS, tabW, Erev, E, zeros):
    mesh = plsc.VectorSubcoreMesh(core_axis_name="c", subcore_axis_name="s")

    @functools.partial(
        pl.kernel,
        mesh=mesh,
        out_type=[jax.ShapeDtypeStruct((2, 2, _NW, _HALF), _f32),
                  jax.ShapeDtypeStruct((2, _NW, _DEGW), _f32)],
        scratch_types=[pltpu.VMEM((_SRCBUF,), jnp.int32),
                       pltpu.VMEM((2, _CHUNK), jnp.int32),
                       pltpu.VMEM((_TAIL,), jnp.int32),
                       pltpu.VMEM((2, _CHUNK, _HALF), _f32),
                       pltpu.VMEM((_CHUNK, _DEGW), _f32),
                       pltpu.VMEM_SHARED((_NW, _HALF), _f32),
                       pltpu.VMEM_SHARED((_NW, _DEGW), _f32),
                       pltpu.SemaphoreType.DMA,
                       pltpu.SemaphoreType.DMA,
                       pltpu.SemaphoreType.DMA,
                       pltpu.SemaphoreType.DMA],
        compiler_params=pltpu.CompilerParams(use_tc_tiling_on_sc=False),
    )
    def body(tabS_h, tabW_h, erev_h, e_h, zro_h, out_h, dout_h,
             srcv, dstv, tailv, rows, onesv, acc, dacc,
             semg, semd, sems, semo):
        c = lax.axis_index("c")
        s = lax.axis_index("s")
        rbase = s * _RPT
        ebase = s * _EPT

        onev = jnp.ones((16,), _f32)

        def fill(i, carry):
            onesv[i, pl.ds(0, 16)] = onev
            return carry

        lax.fori_loop(0, _CHUNK, fill, 0)

        for t in range(2):
            tab_h = tabS_h if t == 0 else tabW_h
            edge_h = erev_h if t == 0 else e_h
            tab_c = tab_h.at[pl.ds(c * _NW, _NW)]

            pltpu.sync_copy(zro_h, acc.at[pl.ds(rbase, _RPT)])
            pltpu.sync_copy(zro_h.at[:, pl.ds(0, _DEGW)],
                            dacc.at[pl.ds(rbase, _RPT)])
            plsc.subcore_barrier()

            def start_chunk(jloc, g, buf):
                pltpu.async_copy(
                    tab_c.at[srcv.at[pl.ds(jloc * _CHUNK, _CHUNK)]],
                    rows.at[buf], semg)
                pltpu.async_copy(
                    edge_h.at[1, pl.ds(ebase + g * _CHUNK, _CHUNK)],
                    dstv.at[buf], semd)

            def drain_chunk(buf):
                pltpu.make_async_copy(tab_h.at[pl.ds(0, _CHUNK)],
                                      rows.at[buf], semg).wait()
                pltpu.make_async_copy(edge_h.at[1, pl.ds(0, _CHUNK)],
                                      dstv.at[buf], semd).wait()

            def start_scatters(buf):
                pltpu.async_copy(rows.at[buf], acc.at[dstv.at[buf]],
                                 sems, add=True)
                pltpu.async_copy(onesv, dacc.at[dstv.at[buf]],
                                 semo, add=True)

            def drain_scatters():
                pltpu.make_async_copy(tab_h.at[pl.ds(0, _CHUNK)],
                                      rows.at[0], sems).wait()
                pltpu.make_async_copy(
                    zro_h.at[pl.ds(0, _CHUNK), pl.ds(0, _DEGW)],
                    onesv, semo).wait()

            for phase in range(2):
                gbase = phase * _PCHUNK
                npre = _PCHUNK * _CHUNK + (_TAIL if phase == 1 else 0)
                pltpu.sync_copy(
                    edge_h.at[0, pl.ds(ebase + gbase * _CHUNK, npre)],
                    srcv.at[pl.ds(0, npre)])

                start_chunk(0, gbase, 0)

                def pair(m, carry):
                    j = 2 * m
                    drain_chunk(0)

                    @pl.when(m > 0)
                    def _():
                        drain_scatters()

                    start_chunk(j + 1, gbase + j + 1, 1)
                    start_scatters(0)
                    drain_chunk(1)
                    drain_scatters()
                    start_chunk(j + 2, gbase + j + 2, 0)
                    start_scatters(1)
                    return carry

                lax.fori_loop(0, (_PCHUNK - 1) // 2, pair, 0)
                drain_chunk(0)
                drain_scatters()
                pltpu.sync_copy(rows.at[0], acc.at[dstv.at[0]], add=True)
                pltpu.sync_copy(onesv, dacc.at[dstv.at[0]], add=True)

            pltpu.sync_copy(
                edge_h.at[1, pl.ds(ebase + _NFULL * _CHUNK, _TAIL)], tailv)
            pltpu.async_copy(
                tab_c.at[srcv.at[pl.ds(_PCHUNK * _CHUNK, _TAIL)]],
                rows.at[0, pl.ds(0, _TAIL)], semg)
            pltpu.make_async_copy(tab_h.at[pl.ds(0, _TAIL)],
                                  rows.at[0, pl.ds(0, _TAIL)], semg).wait()
            pltpu.sync_copy(rows.at[0, pl.ds(0, _TAIL)], acc.at[tailv],
                            add=True)
            pltpu.sync_copy(onesv.at[pl.ds(0, _TAIL)], dacc.at[tailv],
                            add=True)

            plsc.subcore_barrier()
            pltpu.sync_copy(acc.at[pl.ds(rbase, _RPT)],
                            out_h.at[t, c, pl.ds(rbase, _RPT)])

            @pl.when(c == 0)
            def _():
                pltpu.sync_copy(dacc.at[pl.ds(rbase, _RPT)],
                                dout_h.at[t, pl.ds(rbase, _RPT)])
            plsc.subcore_barrier()

    return body(tabS, tabW, Erev, E, zeros)


_TCR = 1000


def _tc_body(xw, xs, aWA, aWB, aSA, aSB, dW, dS,
             W1, b1, W2, b2, Wg1, bg1, Wg2, bg2, W3, b3, W4, b4,
             outw, outs):
    def matT(x, w):
        return lax.dot_general(x, w[:], (((1,), (1,)), ((), ())),
                               preferred_element_type=_f32)

    hw = matT(xw[:], W1) + b1[:][None, :]
    hs = matT(xs[:], W2) + b2[:][None, :]

    sumS = jnp.concatenate([aWA[0, 0], aWB[0, 0]], axis=1)
    degw = dW[0, :, 0:1]
    tw = matT(matT(sumS, W2), Wg1)
    bias_w = matT(b2[:][None, :], Wg1) + bg1[:][None, :]
    nhw = (tw + degw * bias_w) / jnp.maximum(degw, 1.0)
    uw = matT(nhw + hw, W4) + b4[:][None, :]
    mw = jnp.max(uw, axis=1, keepdims=True)
    outw[:] = uw - (mw + jnp.log(jnp.sum(jnp.exp(uw - mw), axis=1,
                                         keepdims=True)))

    sumW = jnp.concatenate([aSA[0, 0], aSB[0, 0]], axis=1)
    degs = dS[0, :, 0:1]
    ts = matT(matT(sumW, W1), Wg2)
    bias_s = matT(b1[:][None, :], Wg2) + bg2[:][None, :]
    nhs = (ts + degs * bias_s) / jnp.maximum(degs, 1.0)
    us = matT(nhs + hs, W3) + b3[:][None, :]
    ms = jnp.max(us, axis=1, keepdims=True)
    outs[:] = us - (ms + jnp.log(jnp.sum(jnp.exp(us - ms), axis=1,
                                         keepdims=True)))


def _tc_dense(Xw, Xs, Agg, Deg,
              W1, b1, W2, b2, Wg1, bg1, Wg2, bg2, W3, b3, W4, b4):
    grid = (_NW // _TCR,)
    row_spec = pl.BlockSpec((_TCR, _D), lambda i: (i, 0))
    w_spec = pl.BlockSpec((_D, _D), lambda i: (0, 0))
    b_spec = pl.BlockSpec((_D,), lambda i: (0,))

    def agg_spec(t, k):
        return pl.BlockSpec((1, 1, _TCR, _HALF),
                            lambda i, t=t, k=k: (t, k, i, 0))

    def deg_spec(t):
        return pl.BlockSpec((1, _TCR, _DEGW), lambda i, t=t: (t, i, 0))

    return pl.pallas_call(
        _tc_body,
        grid=grid,
        in_specs=[row_spec, row_spec,
                  agg_spec(0, 0), agg_spec(0, 1), agg_spec(1, 0),
                  agg_spec(1, 1), deg_spec(0), deg_spec(1),
                  w_spec, b_spec, w_spec, b_spec, w_spec, b_spec,
                  w_spec, b_spec, w_spec, b_spec, w_spec, b_spec],
        out_specs=[row_spec, row_spec],
        out_shape=[jax.ShapeDtypeStruct((_NW, _D), _f32),
                   jax.ShapeDtypeStruct((_NS, _D), _f32)],
    )(Xw, Xs, Agg, Agg, Agg, Agg, Deg, Deg,
      W1, b1, W2, b2, Wg1, bg1, Wg2, bg2, W3, b3, W4, b4)


def kernel(Xw, Xs, E, Erev, W1, b1, W2, b2, Wg1, bg1, Wg2, bg2, W3, b3, W4, b4):
    tabS = jnp.concatenate([Xs[:, :_HALF], Xs[:, _HALF:]], axis=0)
    tabW = jnp.concatenate([Xw[:, :_HALF], Xw[:, _HALF:]], axis=0)
    zeros = jnp.zeros((_RPT, _HALF), _f32)

    Agg, Deg = _sc_aggregate(tabS, tabW, Erev.astype(jnp.int32),
                             E.astype(jnp.int32), zeros)

    ow, os_ = _tc_dense(Xw, Xs, Agg, Deg,
                        W1, b1, W2, b2, Wg1, bg1, Wg2, bg2, W3, b3, W4, b4)
    return (ow, os_)
